# baseline (speedup 1.0000x reference)
import functools

import jax
import jax.numpy as jnp
from jax import lax
from jax.experimental import pallas as pl
from jax.experimental.pallas import tpu as pltpu
from jax.experimental.pallas import tpu_sc as plsc

EMBED_D = 64
NUM_WORKERS = 32
B_TOTAL = 4096 * 200
B_PER_W = B_TOTAL // NUM_WORKERS
CHUNK = 512
NCHUNK = B_PER_W // CHUNK

_mesh = plsc.VectorSubcoreMesh(core_axis_name="c", subcore_axis_name="s")


@functools.partial(
    pl.kernel,
    mesh=_mesh,
    out_type=jax.ShapeDtypeStruct((B_TOTAL, EMBED_D), jnp.float32),
    scratch_types=[
        pltpu.VMEM((CHUNK,), jnp.int32),
        pltpu.VMEM((CHUNK, EMBED_D), jnp.float32),
        pltpu.SemaphoreType.DMA,
    ],
    compiler_params=pltpu.CompilerParams(use_tc_tiling_on_sc=False),
)
def _gather_kernel(table_hbm, idx_hbm, out_hbm, idx_v, rows_v, sem):
    wid = lax.axis_index("s") * 2 + lax.axis_index("c")
    base = wid * B_PER_W

    def body(g, _):
        off = base + g * CHUNK
        pltpu.sync_copy(idx_hbm.at[pl.ds(off, CHUNK)], idx_v)
        pltpu.async_copy(table_hbm.at[idx_v], rows_v, sem).wait()
        pltpu.sync_copy(rows_v, out_hbm.at[pl.ds(off, CHUNK)])
        return ()

    lax.fori_loop(0, NCHUNK, body, ())


def kernel(x, embeddings):
    idx = x.reshape(-1).astype(jnp.int32)
    out = _gather_kernel(embeddings, idx)
    return out.reshape(x.shape[0], x.shape[1], EMBED_D)


# speedup vs baseline: 1.0428x; 1.0428x over previous
import functools

import jax
import jax.numpy as jnp
from jax import lax
from jax.experimental import pallas as pl
from jax.experimental.pallas import tpu as pltpu
from jax.experimental.pallas import tpu_sc as plsc

EMBED_D = 64
NUM_WORKERS = 32
B_TOTAL = 4096 * 200
B_PER_W = B_TOTAL // NUM_WORKERS
CHUNK = 512
NB = 2
NCHUNK = B_PER_W // CHUNK
NOUTER = NCHUNK // NB

_mesh = plsc.VectorSubcoreMesh(core_axis_name="c", subcore_axis_name="s")


@functools.partial(
    pl.kernel,
    mesh=_mesh,
    out_type=jax.ShapeDtypeStruct((B_TOTAL, EMBED_D), jnp.float32),
    scratch_types=[
        pltpu.VMEM((B_PER_W,), jnp.int32),
        [pltpu.VMEM((CHUNK, EMBED_D), jnp.float32) for _ in range(NB)],
        [pltpu.SemaphoreType.DMA for _ in range(NB)],
        [pltpu.SemaphoreType.DMA for _ in range(NB)],
    ],
    compiler_params=pltpu.CompilerParams(use_tc_tiling_on_sc=False),
)
def _gather_kernel(table_hbm, idx_hbm, out_hbm, idx_v, rows, gsem, ssem):
    wid = lax.axis_index("s") * 2 + lax.axis_index("c")
    base = wid * B_PER_W
    pltpu.sync_copy(idx_hbm.at[pl.ds(base, B_PER_W)], idx_v)

    def gather(g, b):
        return pltpu.make_async_copy(
            table_hbm.at[idx_v.at[pl.ds(g * CHUNK, CHUNK)]], rows[b], gsem[b])

    def store(g, b):
        return pltpu.make_async_copy(
            rows[b], out_hbm.at[pl.ds(base + g * CHUNK, CHUNK)], ssem[b])

    def body(p, _):
        g0 = p * NB
        for b in range(NB):
            @pl.when(p > 0)
            def _():
                store(g0 + b - NB, b).wait()
            gather(g0 + b, b).start()
        for b in range(NB):
            gather(g0 + b, b).wait()
            store(g0 + b, b).start()
        return ()

    lax.fori_loop(0, NOUTER, body, ())
    for b in range(NB):
        store(NCHUNK - NB + b, b).wait()


def kernel(x, embeddings):
    idx = x.reshape(-1).astype(jnp.int32)
    out = _gather_kernel(embeddings, idx)
    return out.reshape(x.shape[0], x.shape[1], EMBED_D)


# speedup vs baseline: 1.0483x; 1.0052x over previous
import functools

import jax
import jax.numpy as jnp
from jax import lax
from jax.experimental import pallas as pl
from jax.experimental.pallas import tpu as pltpu
from jax.experimental.pallas import tpu_sc as plsc

EMBED_D = 64
BATCH = 4096
SEQ = 200
NUM_WORKERS = 32
ROWS_PER_W = BATCH // NUM_WORKERS
NB = 4
NGROUP = ROWS_PER_W // NB

_mesh = plsc.VectorSubcoreMesh(core_axis_name="c", subcore_axis_name="s")


@functools.partial(
    pl.kernel,
    mesh=_mesh,
    out_type=jax.ShapeDtypeStruct((BATCH, SEQ, EMBED_D), jnp.float32),
    scratch_types=[
        pltpu.VMEM((ROWS_PER_W, SEQ), jnp.int32),
        [pltpu.VMEM((SEQ, EMBED_D), jnp.float32) for _ in range(NB)],
        [pltpu.SemaphoreType.DMA for _ in range(NB)],
        [pltpu.SemaphoreType.DMA for _ in range(NB)],
    ],
    compiler_params=pltpu.CompilerParams(use_tc_tiling_on_sc=False),
)
def _gather_kernel(table_hbm, x_hbm, out_hbm, idx_v, rows, gsem, ssem):
    wid = lax.axis_index("s") * 2 + lax.axis_index("c")
    base = wid * ROWS_PER_W
    pltpu.sync_copy(x_hbm.at[pl.ds(base, ROWS_PER_W)], idx_v)

    def gather(r, b):
        return pltpu.make_async_copy(table_hbm.at[idx_v.at[r]], rows[b], gsem[b])

    def store(r, b):
        return pltpu.make_async_copy(rows[b], out_hbm.at[base + r], ssem[b])

    def body(p, _):
        r0 = p * NB
        for b in range(NB):
            @pl.when(p > 0)
            def _():
                store(r0 + b - NB, b).wait()
            gather(r0 + b, b).start()
        for b in range(NB):
            gather(r0 + b, b).wait()
            store(r0 + b, b).start()
        return ()

    lax.fori_loop(0, NGROUP, body, ())
    for b in range(NB):
        store(ROWS_PER_W - NB + b, b).wait()


def kernel(x, embeddings):
    return _gather_kernel(embeddings, x)
